# baseline (speedup 1.0000x reference)
import functools

import jax
import jax.numpy as jnp
from jax import lax
from jax.experimental import pallas as pl
from jax.experimental.pallas import tpu as pltpu
from jax.experimental.pallas import tpu_sc as plsc

N = 10000
E = 320000
D = 128
NP = 10240
NS = 16
NW = 2 * NS
EPT = E // NW
CH = 80
NCHUNK = 128
NBLOCK = NCHUNK // 4
CEPT = NCHUNK * CH
RPT = NP // NS
BLK = 512
NB = NP // BLK

_mesh = plsc.VectorSubcoreMesh(core_axis_name="c", subcore_axis_name="s")


@functools.partial(
    pl.kernel,
    out_type=jax.ShapeDtypeStruct((2 * NP, D), jnp.float32),
    mesh=_mesh,
    scratch_types=[
        pltpu.VMEM((8, CH), jnp.int32),
        pltpu.VMEM((CH, D), jnp.float32),
        pltpu.VMEM_SHARED((NP, D), jnp.float32),
        pltpu.SemaphoreType.DMA,
    ],
)
def _segsum_sc(y_hbm, edges_hbm, out_hbm, idx_v, rows_v, acc_sh, sem):
    c = lax.axis_index("c")
    s = lax.axis_index("s")
    r0 = s * RPT
    gbase = (c * NS + s) * NBLOCK
    pltpu.sync_copy(y_hbm.at[pl.ds(r0, RPT)], acc_sh.at[pl.ds(r0, RPT)])
    plsc.subcore_barrier()

    def body(g, carry):
        pltpu.sync_copy(edges_hbm.at[gbase + g], idx_v)
        for u in range(4):
            pltpu.async_copy(y_hbm.at[idx_v.at[2 * u]], rows_v, sem).wait()
            pltpu.sync_copy(rows_v, acc_sh.at[idx_v.at[2 * u + 1]], add=True)
        return carry

    lax.fori_loop(0, NBLOCK, body, 0)
    plsc.subcore_barrier()
    pltpu.sync_copy(acc_sh.at[pl.ds(r0, RPT)], out_hbm.at[pl.ds(c * NP + r0, RPT)])


def _mm_body(x_ref, w_ref, o_ref):
    o_ref[...] = lax.dot_general(
        x_ref[...], w_ref[...], (((1,), (1,)), ((), ())),
        preferred_element_type=jnp.float32)


def _relu_mm_body(sa_ref, sb_ref, y_ref, b_ref, w_ref, o_ref):
    h = jnp.maximum(sa_ref[...] + sb_ref[...] - y_ref[...] + b_ref[...], 0.0)
    o_ref[...] = lax.dot_general(
        h, w_ref[...], (((1,), (1,)), ((), ())),
        preferred_element_type=jnp.float32)


def _final_body(sa_ref, sb_ref, y_ref, b_ref, o_ref):
    o_ref[...] = sa_ref[...] + sb_ref[...] - y_ref[...] + b_ref[...]


def kernel(x, adj, W1, b1, W2, b2):
    srcw = jnp.pad(adj[0].reshape(NW, EPT), ((0, 0), (0, CEPT - EPT)))
    dstw = jnp.pad(adj[1].reshape(NW, EPT), ((0, 0), (0, CEPT - EPT)),
                   constant_values=NP - 1)
    edges8 = jnp.stack(
        [srcw.reshape(NW, NBLOCK, 4, CH), dstw.reshape(NW, NBLOCK, 4, CH)],
        axis=3).reshape(NW * NBLOCK, 8, CH)
    x_pad = jnp.pad(x, ((0, NP - N), (0, 0)))

    y1 = pl.pallas_call(
        _mm_body,
        grid=(NB,),
        in_specs=[
            pl.BlockSpec((BLK, D), lambda j: (j, 0)),
            pl.BlockSpec((D, D), lambda j: (0, 0)),
        ],
        out_specs=pl.BlockSpec((BLK, D), lambda j: (j, 0)),
        out_shape=jax.ShapeDtypeStruct((NP, D), jnp.float32),
    )(x_pad, W1)

    s1 = _segsum_sc(y1, edges8)

    y2 = pl.pallas_call(
        _relu_mm_body,
        grid=(NB,),
        in_specs=[
            pl.BlockSpec((BLK, D), lambda j: (j, 0)),
            pl.BlockSpec((BLK, D), lambda j: (NB + j, 0)),
            pl.BlockSpec((BLK, D), lambda j: (j, 0)),
            pl.BlockSpec((1, D), lambda j: (0, 0)),
            pl.BlockSpec((D, D), lambda j: (0, 0)),
        ],
        out_specs=pl.BlockSpec((BLK, D), lambda j: (j, 0)),
        out_shape=jax.ShapeDtypeStruct((NP, D), jnp.float32),
    )(s1, s1, y1, b1.reshape(1, D), W2)

    s2 = _segsum_sc(y2, edges8)

    out = pl.pallas_call(
        _final_body,
        grid=(NB,),
        in_specs=[
            pl.BlockSpec((BLK, D), lambda j: (j, 0)),
            pl.BlockSpec((BLK, D), lambda j: (NB + j, 0)),
            pl.BlockSpec((BLK, D), lambda j: (j, 0)),
            pl.BlockSpec((1, D), lambda j: (0, 0)),
        ],
        out_specs=pl.BlockSpec((BLK, D), lambda j: (j, 0)),
        out_shape=jax.ShapeDtypeStruct((NP, D), jnp.float32),
    )(s2, s2, y2, b2.reshape(1, D))

    return out[:N]


# speedup vs baseline: 1.7146x; 1.7146x over previous
import functools

import jax
import jax.numpy as jnp
from jax import lax
from jax.experimental import pallas as pl
from jax.experimental.pallas import tpu as pltpu
from jax.experimental.pallas import tpu_sc as plsc

N = 10000
E = 320000
D = 128
NP = 10240
NS = 16
NW = 2 * NS
EPT = E // NW
CH = 80
NCHUNK = 125
RPT = NP // NS
BLK = 512
NB = NP // BLK

_mesh = plsc.VectorSubcoreMesh(core_axis_name="c", subcore_axis_name="s")


@functools.partial(
    pl.kernel,
    out_type=jax.ShapeDtypeStruct((2 * NP, D), jnp.float32),
    mesh=_mesh,
    scratch_types=[
        pltpu.VMEM((8, CH), jnp.int32),
        pltpu.VMEM((CH, D), jnp.float32),
        pltpu.VMEM_SHARED((NP, D), jnp.float32),
        pltpu.SemaphoreType.DMA,
    ],
)
def _segsum_sc(y_hbm, edges_hbm, out_hbm, idx_v, rows_v, acc_sh, sem):
    c = lax.axis_index("c")
    s = lax.axis_index("s")
    r0 = s * RPT
    kbase = (c * NS + s) * NCHUNK
    pltpu.sync_copy(y_hbm.at[pl.ds(r0, RPT)], acc_sh.at[pl.ds(r0, RPT)])
    plsc.subcore_barrier()

    def body(k, carry):
        pltpu.sync_copy(edges_hbm.at[kbase + k], idx_v)
        pltpu.async_copy(y_hbm.at[idx_v.at[0]], rows_v, sem).wait()
        pltpu.sync_copy(rows_v, acc_sh.at[idx_v.at[1]], add=True)
        return carry

    lax.fori_loop(0, NCHUNK, body, 0)
    plsc.subcore_barrier()
    pltpu.sync_copy(acc_sh.at[pl.ds(r0, RPT)], out_hbm.at[pl.ds(c * NP + r0, RPT)])


def _mm_body(x_ref, w_ref, o_ref):
    o_ref[...] = lax.dot_general(
        x_ref[...], w_ref[...], (((1,), (1,)), ((), ())),
        preferred_element_type=jnp.float32)


def _relu_mm_body(sa_ref, sb_ref, y_ref, b_ref, w_ref, o_ref):
    h = jnp.maximum(sa_ref[...] + sb_ref[...] - y_ref[...] + b_ref[...], 0.0)
    o_ref[...] = lax.dot_general(
        h, w_ref[...], (((1,), (1,)), ((), ())),
        preferred_element_type=jnp.float32)


def _final_body(sa_ref, sb_ref, y_ref, b_ref, o_ref):
    o_ref[...] = sa_ref[...] + sb_ref[...] - y_ref[...] + b_ref[...]


def kernel(x, adj, W1, b1, W2, b2):
    edges8 = jnp.pad(
        jnp.stack([adj[0].reshape(NW, NCHUNK, CH),
                   adj[1].reshape(NW, NCHUNK, CH)], axis=2),
        ((0, 0), (0, 0), (0, 6), (0, 0))).reshape(NW * NCHUNK, 8, CH)
    x_pad = jnp.pad(x, ((0, NP - N), (0, 0)))

    y1 = pl.pallas_call(
        _mm_body,
        grid=(NB,),
        in_specs=[
            pl.BlockSpec((BLK, D), lambda j: (j, 0)),
            pl.BlockSpec((D, D), lambda j: (0, 0)),
        ],
        out_specs=pl.BlockSpec((BLK, D), lambda j: (j, 0)),
        out_shape=jax.ShapeDtypeStruct((NP, D), jnp.float32),
    )(x_pad, W1)

    s1 = _segsum_sc(y1, edges8)

    y2 = pl.pallas_call(
        _relu_mm_body,
        grid=(NB,),
        in_specs=[
            pl.BlockSpec((BLK, D), lambda j: (j, 0)),
            pl.BlockSpec((BLK, D), lambda j: (NB + j, 0)),
            pl.BlockSpec((BLK, D), lambda j: (j, 0)),
            pl.BlockSpec((1, D), lambda j: (0, 0)),
            pl.BlockSpec((D, D), lambda j: (0, 0)),
        ],
        out_specs=pl.BlockSpec((BLK, D), lambda j: (j, 0)),
        out_shape=jax.ShapeDtypeStruct((NP, D), jnp.float32),
    )(s1, s1, y1, b1.reshape(1, D), W2)

    s2 = _segsum_sc(y2, edges8)

    out = pl.pallas_call(
        _final_body,
        grid=(NB,),
        in_specs=[
            pl.BlockSpec((BLK, D), lambda j: (j, 0)),
            pl.BlockSpec((BLK, D), lambda j: (NB + j, 0)),
            pl.BlockSpec((BLK, D), lambda j: (j, 0)),
            pl.BlockSpec((1, D), lambda j: (0, 0)),
        ],
        out_specs=pl.BlockSpec((BLK, D), lambda j: (j, 0)),
        out_shape=jax.ShapeDtypeStruct((NP, D), jnp.float32),
    )(s2, s2, y2, b2.reshape(1, D))

    return out[:N]
